# Initial kernel scaffold; baseline (speedup 1.0000x reference)
import jax
import jax.numpy as jnp
from jax.experimental import pallas as pl


def kernel(x, emb_weight, pos_weight):
    raise NotImplementedError("write your pallas kernel here")


# speedup vs baseline: 2.2399x; 2.2399x over previous
import functools

import jax
import jax.numpy as jnp
from jax import lax
from jax.experimental import pallas as pl
from jax.experimental.pallas import tpu as pltpu
from jax.experimental.pallas import tpu_sc as plsc

_EMBED = 32
_L = 16
_NC = 2
_NS = 16
_NW = _NC * _NS
_CHUNK = 128
_GROUP = 32
_GROUPS_PER_CHUNK = _CHUNK // _GROUP
_N_ROWS = 1024 * 32 * 32
_N_CHUNKS = _N_ROWS // _CHUNK
_CHUNKS_PER_W = _N_CHUNKS // _NW


def _lane_sum(v):
    lanes = lax.iota(jnp.int32, _L)
    dnums = lax.GatherDimensionNumbers(
        offset_dims=(), collapsed_slice_dims=(0,), start_index_map=(0,))
    for sh in (8, 4, 2, 1):
        perm = lax.gather(v, (lanes ^ sh)[:, None], dnums, slice_sizes=(1,),
                          mode=lax.GatherScatterMode.PROMISE_IN_BOUNDS)
        v = v + perm
    return v


def _rsqrt_nr(x):
    i = plsc.bitcast(x, jnp.int32)
    i = jnp.int32(0x5F3759DF) - (i >> 1)
    y = plsc.bitcast(i, jnp.float32)
    for _ in range(3):
        y = y * (jnp.float32(1.5) - jnp.float32(0.5) * x * y * y)
    return y


def _make_sc_kernel():
    mesh = plsc.VectorSubcoreMesh(core_axis_name="c", subcore_axis_name="s")

    @functools.partial(
        pl.kernel,
        mesh=mesh,
        compiler_params=pltpu.CompilerParams(needs_layout_passes=False,
                                             use_tc_tiling_on_sc=False),
        out_type=jax.ShapeDtypeStruct((_N_ROWS, _EMBED), jnp.float32),
        scratch_types=[
            pltpu.VMEM((_CHUNKS_PER_W, _CHUNK), jnp.int32),
            pltpu.VMEM((_CHUNK, _EMBED), jnp.float32),
            pltpu.VMEM((_CHUNK, _EMBED), jnp.float32),
            pltpu.SemaphoreType.DMA,
            pltpu.SemaphoreType.DMA,
        ],
    )
    def sc_kernel(idx_hbm, emb_hbm, pos_hbm, out_hbm, idx_v, ea, pb, sem_a, sem_b):
        wid = lax.axis_index("s") * _NC + lax.axis_index("c")
        chunk0 = wid * _CHUNKS_PER_W
        pltpu.sync_copy(idx_hbm.at[pl.ds(chunk0, _CHUNKS_PER_W)], idx_v)

        def chunk_body(c, carry):
            row0 = (chunk0 + c) * _CHUNK
            cp_a = pltpu.async_copy(emb_hbm.at[idx_v.at[c]], ea, sem_a)
            cp_b = pltpu.async_copy(pos_hbm.at[idx_v.at[c]], pb, sem_b)
            cp_a.wait()
            cp_b.wait()

            for g in range(_GROUPS_PER_CHUNK):
                def pass1(r, acc):
                    s, ss = acc
                    row = g * _GROUP + r
                    y0 = ea[row, pl.ds(0, _L)] + pb[row, pl.ds(0, _L)]
                    y1 = ea[row, pl.ds(_L, _L)] + pb[row, pl.ds(_L, _L)]
                    ea[row, pl.ds(0, _L)] = y0
                    ea[row, pl.ds(_L, _L)] = y1
                    return s + (y0 + y1), ss + (y0 * y0 + y1 * y1)

                zero = jnp.zeros((_L,), jnp.float32)
                s, ss = lax.fori_loop(0, _GROUP, pass1, (zero, zero))
                inv_n = jnp.float32(1.0 / (_GROUP * _EMBED))
                mean_v = _lane_sum(s) * inv_n
                var_v = jnp.maximum(_lane_sum(ss) * inv_n - mean_v * mean_v,
                                    jnp.float32(0.0))
                scale_v = _rsqrt_nr(var_v + jnp.float32(1e-5))

                def pass2(r, _):
                    row = g * _GROUP + r
                    ea[row, pl.ds(0, _L)] = (ea[row, pl.ds(0, _L)] - mean_v) * scale_v
                    ea[row, pl.ds(_L, _L)] = (ea[row, pl.ds(_L, _L)] - mean_v) * scale_v
                    return 0

                lax.fori_loop(0, _GROUP, pass2, 0)

            pltpu.sync_copy(ea, out_hbm.at[pl.ds(row0, _CHUNK)])
            return carry

        lax.fori_loop(0, _CHUNKS_PER_W, chunk_body, 0)

    return sc_kernel


_sc_kernel = _make_sc_kernel()


def kernel(x, emb_weight, pos_weight):
    b, l, d = x.shape
    e = emb_weight.shape[1]
    idx = x.reshape(_N_CHUNKS, _CHUNK)
    out = _sc_kernel(idx, emb_weight, pos_weight)
    return out.reshape(b, l, d, e)


# speedup vs baseline: 2.8365x; 1.2663x over previous
import functools

import jax
import jax.numpy as jnp
from jax import lax
from jax.experimental import pallas as pl
from jax.experimental.pallas import tpu as pltpu
from jax.experimental.pallas import tpu_sc as plsc

_EMBED = 32
_L = 16
_NC = 2
_NS = 16
_NW = _NC * _NS
_CHUNK = 128
_GROUP = 32
_GROUPS_PER_CHUNK = _CHUNK // _GROUP
_N_ROWS = 1024 * 32 * 32
_N_CHUNKS = _N_ROWS // _CHUNK
_CHUNKS_PER_W = _N_CHUNKS // _NW
_UNROLL = 4


def _lane_sum(v):
    lanes = lax.iota(jnp.int32, _L)
    dnums = lax.GatherDimensionNumbers(
        offset_dims=(), collapsed_slice_dims=(0,), start_index_map=(0,))
    for sh in (8, 4, 2, 1):
        perm = lax.gather(v, (lanes ^ sh)[:, None], dnums, slice_sizes=(1,),
                          mode=lax.GatherScatterMode.PROMISE_IN_BOUNDS)
        v = v + perm
    return v


def _rsqrt_nr(x):
    i = plsc.bitcast(x, jnp.int32)
    i = jnp.int32(0x5F3759DF) - (i >> 1)
    y = plsc.bitcast(i, jnp.float32)
    for _ in range(3):
        y = y * (jnp.float32(1.5) - jnp.float32(0.5) * x * y * y)
    return y


def _compute_chunk(ea, pb, ob):
    for g in range(_GROUPS_PER_CHUNK):
        g0 = g * _GROUP

        def pass1(r, acc):
            s, ss = acc
            for u in range(_UNROLL):
                row = g0 + r * _UNROLL + u
                y0 = ea[row, pl.ds(0, _L)] + pb[row, pl.ds(0, _L)]
                y1 = ea[row, pl.ds(_L, _L)] + pb[row, pl.ds(_L, _L)]
                ob[row, pl.ds(0, _L)] = y0
                ob[row, pl.ds(_L, _L)] = y1
                s = s + (y0 + y1)
                ss = ss + (y0 * y0 + y1 * y1)
            return s, ss

        zero = jnp.zeros((_L,), jnp.float32)
        s, ss = lax.fori_loop(0, _GROUP // _UNROLL, pass1, (zero, zero))
        inv_n = jnp.float32(1.0 / (_GROUP * _EMBED))
        mean_v = _lane_sum(s) * inv_n
        var_v = jnp.maximum(_lane_sum(ss) * inv_n - mean_v * mean_v,
                            jnp.float32(0.0))
        scale_v = _rsqrt_nr(var_v + jnp.float32(1e-5))
        shift_v = mean_v * scale_v

        def pass2(r, carry):
            for u in range(_UNROLL):
                row = g0 + r * _UNROLL + u
                ob[row, pl.ds(0, _L)] = ob[row, pl.ds(0, _L)] * scale_v - shift_v
                ob[row, pl.ds(_L, _L)] = ob[row, pl.ds(_L, _L)] * scale_v - shift_v
            return carry

        lax.fori_loop(0, _GROUP // _UNROLL, pass2, 0)


def _make_sc_kernel():
    mesh = plsc.VectorSubcoreMesh(core_axis_name="c", subcore_axis_name="s")
    f32 = jnp.float32

    @functools.partial(
        pl.kernel,
        mesh=mesh,
        compiler_params=pltpu.CompilerParams(needs_layout_passes=False,
                                             use_tc_tiling_on_sc=False),
        out_type=jax.ShapeDtypeStruct((_N_ROWS, _EMBED), f32),
        scratch_types=[
            pltpu.VMEM((_CHUNKS_PER_W, _CHUNK), jnp.int32),
            pltpu.VMEM((_CHUNK, _EMBED), f32),
            pltpu.VMEM((_CHUNK, _EMBED), f32),
            pltpu.VMEM((_CHUNK, _EMBED), f32),
            pltpu.VMEM((_CHUNK, _EMBED), f32),
            pltpu.VMEM((_CHUNK, _EMBED), f32),
            pltpu.VMEM((_CHUNK, _EMBED), f32),
            pltpu.SemaphoreType.DMA,
            pltpu.SemaphoreType.DMA,
            pltpu.SemaphoreType.DMA,
            pltpu.SemaphoreType.DMA,
        ],
    )
    def sc_kernel(idx_hbm, emb_hbm, pos_hbm, out_hbm, idx_v,
                  ea0, pb0, ob0, ea1, pb1, ob1, sg0, sg1, so0, so1):
        wid = lax.axis_index("s") * _NC + lax.axis_index("c")
        chunk0 = wid * _CHUNKS_PER_W
        bufs = ((ea0, pb0, ob0, sg0, so0), (ea1, pb1, ob1, sg1, so1))

        pltpu.sync_copy(idx_hbm.at[pl.ds(chunk0, _CHUNKS_PER_W)], idx_v)

        def issue_gathers(c, ea, pb, sg):
            pltpu.async_copy(emb_hbm.at[idx_v.at[c]], ea, sg)
            pltpu.async_copy(pos_hbm.at[idx_v.at[c]], pb, sg)

        for b in (0, 1):
            ea, pb, _, sg, _ = bufs[b]
            issue_gathers(b, ea, pb, sg)

        n_iter = _CHUNKS_PER_W // 2

        def body(j, carry):
            for b in (0, 1):
                ea, pb, ob, sg, so = bufs[b]
                c = j * 2 + b
                pltpu.make_async_copy(emb_hbm.at[pl.ds(0, _CHUNK)], ea, sg).wait()
                pltpu.make_async_copy(pos_hbm.at[pl.ds(0, _CHUNK)], pb, sg).wait()

                @pl.when(j > 0)
                def _():
                    pltpu.make_async_copy(
                        ob, out_hbm.at[pl.ds(0, _CHUNK)], so).wait()

                _compute_chunk(ea, pb, ob)

                row0 = (chunk0 + c) * _CHUNK
                pltpu.async_copy(ob, out_hbm.at[pl.ds(row0, _CHUNK)], so)

                @pl.when(j < n_iter - 1)
                def _():
                    issue_gathers(c + 2, ea, pb, sg)
            return carry

        lax.fori_loop(0, n_iter, body, 0)

        for b in (0, 1):
            _, _, ob, _, so = bufs[b]
            pltpu.make_async_copy(ob, out_hbm.at[pl.ds(0, _CHUNK)], so).wait()

    return sc_kernel


_sc_kernel = _make_sc_kernel()


def kernel(x, emb_weight, pos_weight):
    b, l, d = x.shape
    e = emb_weight.shape[1]
    idx = x.reshape(_N_CHUNKS, _CHUNK)
    out = _sc_kernel(idx, emb_weight, pos_weight)
    return out.reshape(b, l, d, e)


# speedup vs baseline: 3.4915x; 1.2309x over previous
import functools

import jax
import jax.numpy as jnp
from jax import lax
from jax.experimental import pallas as pl
from jax.experimental.pallas import tpu as pltpu
from jax.experimental.pallas import tpu_sc as plsc

_EMBED = 32
_L = 16
_NC = 2
_NS = 16
_NW = _NC * _NS
_CHUNK = 128
_GROUP = 32
_GROUPS_PER_CHUNK = _CHUNK // _GROUP
_N_ROWS = 1024 * 32 * 32
_N_CHUNKS = _N_ROWS // _CHUNK
_CHUNKS_PER_W = _N_CHUNKS // _NW
_UNROLL = 4


def _lane_sum(v):
    lanes = lax.iota(jnp.int32, _L)
    dnums = lax.GatherDimensionNumbers(
        offset_dims=(), collapsed_slice_dims=(0,), start_index_map=(0,))
    for sh in (8, 4, 2, 1):
        perm = lax.gather(v, (lanes ^ sh)[:, None], dnums, slice_sizes=(1,),
                          mode=lax.GatherScatterMode.PROMISE_IN_BOUNDS)
        v = v + perm
    return v


def _rsqrt_nr(x):
    i = plsc.bitcast(x, jnp.int32)
    i = jnp.int32(0x5F3759DF) - (i >> 1)
    y = plsc.bitcast(i, jnp.float32)
    for _ in range(3):
        y = y * (jnp.float32(1.5) - jnp.float32(0.5) * x * y * y)
    return y


def _compute_chunk(ea, ob):
    for g in range(_GROUPS_PER_CHUNK):
        g0 = g * _GROUP

        def pass1(r, acc):
            s, ss = acc
            for u in range(_UNROLL):
                row = g0 + r * _UNROLL + u
                y0 = ea[row, pl.ds(0, _L)]
                y1 = ea[row, pl.ds(_L, _L)]
                s = s + (y0 + y1)
                ss = ss + (y0 * y0 + y1 * y1)
            return s, ss

        zero = jnp.zeros((_L,), jnp.float32)
        s, ss = lax.fori_loop(0, _GROUP // _UNROLL, pass1, (zero, zero))
        inv_n = jnp.float32(1.0 / (_GROUP * _EMBED))
        mean_v = _lane_sum(s) * inv_n
        var_v = jnp.maximum(_lane_sum(ss) * inv_n - mean_v * mean_v,
                            jnp.float32(0.0))
        scale_v = _rsqrt_nr(var_v + jnp.float32(1e-5))
        shift_v = mean_v * scale_v

        def pass2(r, carry):
            for u in range(_UNROLL):
                row = g0 + r * _UNROLL + u
                ob[row, pl.ds(0, _L)] = ea[row, pl.ds(0, _L)] * scale_v - shift_v
                ob[row, pl.ds(_L, _L)] = ea[row, pl.ds(_L, _L)] * scale_v - shift_v
            return carry

        lax.fori_loop(0, _GROUP // _UNROLL, pass2, 0)


def _make_sc_kernel():
    mesh = plsc.VectorSubcoreMesh(core_axis_name="c", subcore_axis_name="s")
    f32 = jnp.float32

    @functools.partial(
        pl.kernel,
        mesh=mesh,
        compiler_params=pltpu.CompilerParams(needs_layout_passes=False,
                                             use_tc_tiling_on_sc=False),
        out_type=jax.ShapeDtypeStruct((_N_ROWS, _EMBED), f32),
        scratch_types=[
            pltpu.VMEM((_CHUNKS_PER_W, _CHUNK), jnp.int32),
            pltpu.VMEM((_CHUNK, _EMBED), f32),
            pltpu.VMEM((_CHUNK, _EMBED), f32),
            pltpu.VMEM((_CHUNK, _EMBED), f32),
            pltpu.VMEM((_CHUNK, _EMBED), f32),
            pltpu.SemaphoreType.DMA,
            pltpu.SemaphoreType.DMA,
            pltpu.SemaphoreType.DMA,
            pltpu.SemaphoreType.DMA,
        ],
    )
    def sc_kernel(idx_hbm, tab_hbm, out_hbm, idx_v,
                  ea0, ob0, ea1, ob1, sg0, sg1, so0, so1):
        wid = lax.axis_index("s") * _NC + lax.axis_index("c")
        chunk0 = wid * _CHUNKS_PER_W
        bufs = ((ea0, ob0, sg0, so0), (ea1, ob1, sg1, so1))

        pltpu.sync_copy(idx_hbm.at[pl.ds(chunk0, _CHUNKS_PER_W)], idx_v)

        for b in (0, 1):
            ea, _, sg, _ = bufs[b]
            pltpu.async_copy(tab_hbm.at[idx_v.at[b]], ea, sg)

        n_iter = _CHUNKS_PER_W // 2

        def body(j, carry):
            for b in (0, 1):
                ea, ob, sg, so = bufs[b]
                c = j * 2 + b
                pltpu.make_async_copy(tab_hbm.at[pl.ds(0, _CHUNK)], ea, sg).wait()

                @pl.when(j > 0)
                def _():
                    pltpu.make_async_copy(
                        ob, out_hbm.at[pl.ds(0, _CHUNK)], so).wait()

                _compute_chunk(ea, ob)

                row0 = (chunk0 + c) * _CHUNK
                pltpu.async_copy(ob, out_hbm.at[pl.ds(row0, _CHUNK)], so)

                @pl.when(j < n_iter - 1)
                def _():
                    pltpu.async_copy(tab_hbm.at[idx_v.at[c + 2]], ea, sg)
            return carry

        lax.fori_loop(0, n_iter, body, 0)

        for b in (0, 1):
            _, ob, _, so = bufs[b]
            pltpu.make_async_copy(ob, out_hbm.at[pl.ds(0, _CHUNK)], so).wait()

    return sc_kernel


_sc_kernel = _make_sc_kernel()


def kernel(x, emb_weight, pos_weight):
    b, l, d = x.shape
    e = emb_weight.shape[1]
    tab = emb_weight + pos_weight
    idx = x.reshape(_N_CHUNKS, _CHUNK)
    out = _sc_kernel(idx, tab)
    return out.reshape(b, l, d, e)
